# baseline (speedup 1.0000x reference)
import jax
import jax.numpy as jnp
from jax.experimental import pallas as pl
from jax.experimental.pallas import tpu as pltpu

_COUNTS = 12
_ROWS = 36
_TOK = 77
_DIM = 512


def _body(pos_ref, f_ref, h_ref, a_ref, out_ref):
    pos0 = pos_ref[0]
    pos1 = pos_ref[1]
    f = f_ref[...]
    out_ref[...] = jnp.broadcast_to(f, (4, _TOK, _DIM))
    out_ref[:, pl.ds(pos0, 1), :] = jnp.broadcast_to(h_ref[...], (4, 1, _DIM))
    out_ref[:, pl.ds(pos1, 1), :] = a_ref[...]


def kernel(freeze_embedding, height_param, angle_param, pos0, pos1):
    posv = jnp.stack([jnp.asarray(pos0, jnp.int32),
                      jnp.asarray(pos1, jnp.int32)])
    return pl.pallas_call(
        _body,
        grid=(3, 3),
        in_specs=[
            pl.BlockSpec(memory_space=pltpu.SMEM),
            pl.BlockSpec((1, _TOK, _DIM), lambda i, j: (i, 0, 0)),
            pl.BlockSpec((1, 1, _DIM), lambda i, j: (j, 0, 0)),
            pl.BlockSpec((4, 1, _DIM), lambda i, j: (0, 0, 0)),
        ],
        out_specs=pl.BlockSpec((4, _TOK, _DIM), lambda i, j: (i * 3 + j, 0, 0)),
        out_shape=jax.ShapeDtypeStruct((_ROWS, _TOK, _DIM), jnp.float32),
    )(posv, freeze_embedding,
      height_param.reshape(3, 1, _DIM), angle_param.reshape(4, 1, _DIM))


# speedup vs baseline: 1.2674x; 1.2674x over previous
import jax
import jax.numpy as jnp
from jax.experimental import pallas as pl
from jax.experimental.pallas import tpu as pltpu

_COUNTS = 12
_ROWS = 36
_TOK = 77
_DIM = 512
_NQ = 4
_CH = _ROWS // _NQ


def _body(pos_ref, f_ref, h_ref, a_ref, out_hbm, buf, sems):
    pos0 = pos_ref[0]
    pos1 = pos_ref[1]
    f = f_ref[...]
    h = h_ref[...]
    a = a_ref[...]
    buf[...] = jnp.broadcast_to(
        f[:, None], (3, _COUNTS, _TOK, _DIM)).reshape(_ROWS, _TOK, _DIM)
    h36 = jnp.broadcast_to(h[None, :, None], (3, 3, 4, 1, _DIM)).reshape(
        _ROWS, 1, _DIM)
    a36 = jnp.broadcast_to(a[None], (9, 4, 1, _DIM)).reshape(_ROWS, 1, _DIM)
    buf[:, pl.ds(pos0, 1), :] = h36
    buf[:, pl.ds(pos1, 1), :] = a36
    copies = [
        pltpu.make_async_copy(
            buf.at[pl.ds(k * _CH, _CH)], out_hbm.at[pl.ds(k * _CH, _CH)],
            sems.at[k])
        for k in range(_NQ)
    ]
    for c in copies:
        c.start()
    for c in copies:
        c.wait()


def kernel(freeze_embedding, height_param, angle_param, pos0, pos1):
    posv = jnp.stack([jnp.asarray(pos0, jnp.int32),
                      jnp.asarray(pos1, jnp.int32)])
    return pl.pallas_call(
        _body,
        in_specs=[
            pl.BlockSpec(memory_space=pltpu.SMEM),
            pl.BlockSpec(memory_space=pltpu.VMEM),
            pl.BlockSpec(memory_space=pltpu.VMEM),
            pl.BlockSpec(memory_space=pltpu.VMEM),
        ],
        out_specs=pl.BlockSpec(memory_space=pl.ANY),
        out_shape=jax.ShapeDtypeStruct((_ROWS, _TOK, _DIM), jnp.float32),
        scratch_shapes=[
            pltpu.VMEM((_ROWS, _TOK, _DIM), jnp.float32),
            pltpu.SemaphoreType.DMA((_NQ,)),
        ],
    )(posv, freeze_embedding,
      height_param.reshape(3, 1, _DIM), angle_param.reshape(4, 1, _DIM))
